# baseline (speedup 1.0000x reference)
import functools

import jax
import jax.numpy as jnp
from jax import lax
from jax.experimental import pallas as pl
from jax.experimental.pallas import tpu as pltpu
from jax.experimental.pallas import tpu_sc as plsc

B = 64
N = 16384
L = 16
NV = N // L
NBINS = 2048
RBINS = N + L
NC = 2
NS = 16
ROWS_PER_W = B // (NC * NS)


def _lane():
    return lax.iota(jnp.int32, L)


def _f2s(bits):
    return jnp.where(bits >= 0, bits, bits ^ jnp.int32(0x7FFFFFFF))


def _s2f(s):
    return plsc.bitcast(jnp.where(s >= 0, s, s ^ jnp.int32(0x7FFFFFFF)),
                        jnp.float32)


def _take(x, idx):
    return jnp.take_along_axis(x, idx, axis=0)


def _radix_sort(src_ref, tmp_ref, hist, pack, dest):
    ones = jnp.ones((L,), jnp.int32)
    zeros = jnp.zeros((L,), jnp.int32)

    bufs = [src_ref, tmp_ref]
    passes = [(0, 0x7FF, 0), (11, 0x7FF, 0), (22, 0x3FF, 0x200)]
    for p, (shift, dmask, flip) in enumerate(passes):
        a, b = bufs[p % 2], bufs[(p + 1) % 2]
        sh = jnp.full((L,), shift, jnp.int32)

        def keyfn(keys):
            return _f2s(keys) if p == 0 else keys

        def digit(keys):
            d = lax.shift_right_logical(keys, sh) & jnp.int32(dmask)
            return d ^ jnp.int32(flip) if flip else d

        @plsc.parallel_loop(0, NBINS // L, unroll=4)
        def _zero(i):
            hist[pl.ds(i * L, L)] = zeros

        @plsc.parallel_loop(0, NV, unroll=4)
        def _pre(v):
            sl = pl.ds(v * L, L)
            dig = digit(keyfn(a[sl]))
            cnt, lastm = plsc.scan_count(dig)
            plsc.addupdate_scatter(hist, [dig], cnt, mask=lastm)
            pack[sl] = (dig | ((cnt - 1) << 11)
                        | (jnp.where(lastm, 1, 0) << 15))

        @plsc.parallel_loop(0, NBINS // L, unroll=4,
                            carry=jnp.zeros((L,), jnp.int32))
        def _scan(i, carry):
            sl = pl.ds(i * L, L)
            h = hist[sl]
            c = plsc.cumsum(h)
            hist[sl] = carry + c - h
            return carry + jnp.full((L,), jnp.sum(h), jnp.int32)

        def dloop(v, _):
            sl = pl.ds(v * L, L)
            pk = pack[sl]
            dig = pk & jnp.int32(0x7FF)
            occ = lax.shift_right_logical(pk, jnp.full((L,), 11, jnp.int32)) \
                & jnp.int32(0xF)
            lastm = pk >= jnp.int32(1 << 15)
            base = plsc.load_gather(hist, [dig])
            dest[sl] = base + occ
            plsc.addupdate_scatter(hist, [dig], occ + 1, mask=lastm)
            return 0

        lax.fori_loop(0, NV, dloop, 0)

        @plsc.parallel_loop(0, NV, unroll=4)
        def _scatter(v):
            sl = pl.ds(v * L, L)
            plsc.store_scatter(b, [dest[sl]], keyfn(a[sl]))


def _y_phase(xs_ref, ys_ref, rbins_ref, xends_ref, mg, acc):
    lane = _lane()
    inv_n = jnp.float32(1.0 / N)

    @plsc.parallel_loop(0, NV, unroll=4, carry=acc)
    def _body(v, acc):
        j = v * L + lane
        yv = ys_ref[pl.ds(v * L, L)]
        d = (lax.shift_right_logical(yv, jnp.full((L,), 22, jnp.int32))
             & jnp.int32(0x3FF)) ^ jnp.int32(0x200)
        hi = plsc.load_gather(xends_ref, [d])
        lo = jnp.where(d > 0,
                       plsc.load_gather(xends_ref, [jnp.maximum(d - 1, 0)]),
                       0)

        def _cond(carry):
            lo, hi = carry
            return jnp.max(hi - lo) > 0

        def _step(carry):
            lo, hi = carry
            mid = (lo + hi) >> 1
            val = plsc.load_gather(xs_ref, [jnp.minimum(mid, N - 1)])
            pred = val <= yv
            return (jnp.where(pred, mid + 1, lo), jnp.where(pred, hi, mid))

        q, _ = lax.while_loop(_cond, _step, (lo, hi))
        ynext = jnp.where(
            j < N - 1,
            _s2f(plsc.load_gather(ys_ref, [jnp.minimum(j + 1, N - 1)])),
            mg)
        xcand = jnp.where(
            q < N,
            _s2f(plsc.load_gather(xs_ref, [jnp.minimum(q, N - 1)])),
            mg)
        nxt = jnp.minimum(ynext, xcand)
        cy = (q - (j + 1)).astype(jnp.float32) * inv_n
        acc = acc + cy * cy * (nxt - _s2f(yv))
        qprev = _take(q, jnp.maximum(lane - 1, 0))
        start = (lane == 0) | (q != qprev)
        startpos = plsc.cummax(jnp.where(start, lane, 0))
        cnt = lane - startpos + 1
        qnext = _take(q, jnp.minimum(lane + 1, L - 1))
        is_last = (lane == L - 1) | (q != qnext)
        plsc.addupdate_scatter(rbins_ref, [q], cnt, mask=is_last)
        return acc

    return _body


def _x_phase(xs_ref, ys_ref, rbins_ref, mg, acc):
    lane = _lane()
    inv_n = jnp.float32(1.0 / N)

    @plsc.parallel_loop(0, NV, unroll=4,
                        carry=(acc, jnp.zeros((L,), jnp.int32)))
    def _body(v, carry):
        acc, rc = carry
        i = v * L + lane
        cnts = rbins_ref[pl.ds(v * L, L)]
        r = plsc.cumsum(cnts) + rc
        rc = rc + jnp.full((L,), jnp.sum(cnts), jnp.int32)
        xv = xs_ref[pl.ds(v * L, L)]
        xnext = jnp.where(
            i < N - 1,
            _s2f(plsc.load_gather(xs_ref, [jnp.minimum(i + 1, N - 1)])),
            mg)
        ycand = jnp.where(
            r < N,
            _s2f(plsc.load_gather(ys_ref, [jnp.minimum(r, N - 1)])),
            mg)
        nxt = jnp.minimum(xnext, ycand)
        cx = (i + 1 - r).astype(jnp.float32) * inv_n
        acc = acc + cx * cx * (nxt - _s2f(xv))
        return (acc, rc)

    acc, _ = _body
    return acc


def _vsqrt(v):
    g = lax.shift_right_logical(plsc.bitcast(v, jnp.int32),
                                jnp.full((L,), 1, jnp.int32))
    y = plsc.bitcast(g + jnp.int32(0x1FBD1DF5), jnp.float32)
    for _ in range(4):
        y = jnp.float32(0.5) * (y + v / y)
    return jnp.where(v > 0, y, jnp.float32(0.0))


@functools.lru_cache(maxsize=None)
def _build():
    mesh = plsc.VectorSubcoreMesh(core_axis_name="c", subcore_axis_name="s")

    @functools.partial(
        pl.kernel,
        out_type=jax.ShapeDtypeStruct((B, L), jnp.float32),
        mesh=mesh,
        compiler_params=pltpu.CompilerParams(needs_layout_passes=False),
        scratch_types=[
            pltpu.VMEM((N,), jnp.int32),
            pltpu.VMEM((N,), jnp.int32),
            pltpu.VMEM((N,), jnp.int32),
            pltpu.VMEM((N,), jnp.int32),
            pltpu.VMEM((NBINS,), jnp.int32),
            pltpu.VMEM((N,), jnp.int32),
            pltpu.VMEM((N,), jnp.int32),
            pltpu.VMEM((RBINS,), jnp.int32),
            pltpu.VMEM((L,), jnp.float32),
        ],
    )
    def dist_kernel(x_hbm, y_hbm, out_hbm, xa, xb, ya, yb, hist, pack, dest, rbins, res):
        wid = lax.axis_index("s") * NC + lax.axis_index("c")
        zeros = jnp.zeros((L,), jnp.int32)

        def row_body(rr, _):
            row = wid * ROWS_PER_W + rr
            pltpu.sync_copy(x_hbm.at[row], xa)
            pltpu.sync_copy(y_hbm.at[row], ya)
            _radix_sort(ya, yb, hist, pack, dest)
            _radix_sort(xa, xb, hist, pack, dest)
            xs, ys = xb, yb

            @plsc.parallel_loop(0, RBINS // L, unroll=4)
            def _zr(i):
                rbins[pl.ds(i * L, L)] = zeros

            ms = jnp.maximum(jnp.max(xs[pl.ds(N - L, L)]),
                             jnp.max(ys[pl.ds(N - L, L)]))
            mg = _s2f(jnp.full((L,), ms, jnp.int32))
            acc = jnp.zeros((L,), jnp.float32)
            acc = _y_phase(xs, ys, rbins, hist, mg, acc)
            acc = _x_phase(xs, ys, rbins, mg, acc)
            res[...] = _vsqrt(jnp.full((L,), jnp.sum(acc), jnp.float32))
            pltpu.sync_copy(res, out_hbm.at[row])
            return 0

        lax.fori_loop(0, ROWS_PER_W, row_body, 0)

    return dist_kernel


def kernel(x_values, y_values):
    xi = lax.bitcast_convert_type(x_values, jnp.int32)
    yi = lax.bitcast_convert_type(y_values, jnp.int32)
    return _build()(xi, yi)[:, 0]


# speedup vs baseline: 2.8354x; 2.8354x over previous
import functools

import jax
import jax.numpy as jnp
from jax import lax
from jax.experimental import pallas as pl
from jax.experimental.pallas import tpu as pltpu
from jax.experimental.pallas import tpu_sc as plsc

B = 64
N = 16384
L = 16
NV = N // L
NBINS = 2048
RBINS = N + L
NC = 2
NS = 16
ROWS_PER_W = B // (NC * NS)


def _lane():
    return lax.iota(jnp.int32, L)


def _f2s(bits):
    return jnp.where(bits >= 0, bits, bits ^ jnp.int32(0x7FFFFFFF))


def _s2f(s):
    return plsc.bitcast(jnp.where(s >= 0, s, s ^ jnp.int32(0x7FFFFFFF)),
                        jnp.float32)


def _take(x, idx):
    return jnp.take_along_axis(x, idx, axis=0)


def _radix_sort(src_ref, tmp_ref, hist, pack, dest):
    ones = jnp.ones((L,), jnp.int32)
    zeros = jnp.zeros((L,), jnp.int32)

    bufs = [src_ref, tmp_ref]
    passes = [(0, 0x7FF, 0), (11, 0x7FF, 0), (22, 0x3FF, 0x200)]
    for p, (shift, dmask, flip) in enumerate(passes):
        a, b = bufs[p % 2], bufs[(p + 1) % 2]
        sh = jnp.full((L,), shift, jnp.int32)

        def keyfn(keys):
            return _f2s(keys) if p == 0 else keys

        def digit(keys):
            d = lax.shift_right_logical(keys, sh) & jnp.int32(dmask)
            return d ^ jnp.int32(flip) if flip else d

        @plsc.parallel_loop(0, NBINS // L, unroll=4)
        def _zero(i):
            hist[pl.ds(i * L, L)] = zeros

        @plsc.parallel_loop(0, NV, unroll=4)
        def _pre(v):
            sl = pl.ds(v * L, L)
            dig = digit(keyfn(a[sl]))
            cnt, lastm = plsc.scan_count(dig)
            plsc.addupdate_scatter(hist, [dig], cnt, mask=lastm)
            pack[sl] = (dig | ((cnt - 1) << 11)
                        | (jnp.where(lastm, 1, 0) << 15))

        @plsc.parallel_loop(0, NBINS // L, unroll=4,
                            carry=jnp.zeros((L,), jnp.int32))
        def _scan(i, carry):
            sl = pl.ds(i * L, L)
            h = hist[sl]
            c = plsc.cumsum(h)
            hist[sl] = carry + c - h
            return carry + jnp.full((L,), jnp.sum(h), jnp.int32)

        def dloop(v, _):
            sl = pl.ds(v * L, L)
            pk = pack[sl]
            dig = pk & jnp.int32(0x7FF)
            occ = lax.shift_right_logical(pk, jnp.full((L,), 11, jnp.int32)) \
                & jnp.int32(0xF)
            lastm = pk >= jnp.int32(1 << 15)
            base = plsc.load_gather(hist, [dig])
            dest[sl] = base + occ
            plsc.addupdate_scatter(hist, [dig], occ + 1, mask=lastm)
            return 0

        lax.fori_loop(0, NV, dloop, 0)

        @plsc.parallel_loop(0, NV, unroll=4)
        def _scatter(v):
            sl = pl.ds(v * L, L)
            plsc.store_scatter(b, [dest[sl]], keyfn(a[sl]))


def _y_phase(xs_ref, ys_ref, rbins_ref, xends_ref, mg, acc):
    lane = _lane()
    inv_n = jnp.float32(1.0 / N)

    @plsc.parallel_loop(0, NV, unroll=4, carry=acc)
    def _body(v, acc):
        j = v * L + lane
        yv = ys_ref[pl.ds(v * L, L)]
        d = (lax.shift_right_logical(yv, jnp.full((L,), 22, jnp.int32))
             & jnp.int32(0x3FF)) ^ jnp.int32(0x200)
        hi = plsc.load_gather(xends_ref, [d])
        lo = jnp.where(d > 0,
                       plsc.load_gather(xends_ref, [jnp.maximum(d - 1, 0)]),
                       0)
        for _ in range(15):
            mid = (lo + hi) >> 1
            val = plsc.load_gather(xs_ref, [jnp.minimum(mid, N - 1)])
            pred = val <= yv
            lo = jnp.where(pred, mid + 1, lo)
            hi = jnp.where(pred, hi, mid)
        q = lo
        ynext = jnp.where(
            j < N - 1,
            _s2f(plsc.load_gather(ys_ref, [jnp.minimum(j + 1, N - 1)])),
            mg)
        xcand = jnp.where(
            q < N,
            _s2f(plsc.load_gather(xs_ref, [jnp.minimum(q, N - 1)])),
            mg)
        nxt = jnp.minimum(ynext, xcand)
        cy = (q - (j + 1)).astype(jnp.float32) * inv_n
        acc = acc + cy * cy * (nxt - _s2f(yv))
        qprev = _take(q, jnp.maximum(lane - 1, 0))
        start = (lane == 0) | (q != qprev)
        startpos = plsc.cummax(jnp.where(start, lane, 0))
        cnt = lane - startpos + 1
        qnext = _take(q, jnp.minimum(lane + 1, L - 1))
        is_last = (lane == L - 1) | (q != qnext)
        plsc.addupdate_scatter(rbins_ref, [q], cnt, mask=is_last)
        return acc

    return _body


def _x_phase(xs_ref, ys_ref, rbins_ref, mg, acc):
    lane = _lane()
    inv_n = jnp.float32(1.0 / N)

    @plsc.parallel_loop(0, NV, unroll=4,
                        carry=(acc, jnp.zeros((L,), jnp.int32)))
    def _body(v, carry):
        acc, rc = carry
        i = v * L + lane
        cnts = rbins_ref[pl.ds(v * L, L)]
        r = plsc.cumsum(cnts) + rc
        rc = rc + jnp.full((L,), jnp.sum(cnts), jnp.int32)
        xv = xs_ref[pl.ds(v * L, L)]
        xnext = jnp.where(
            i < N - 1,
            _s2f(plsc.load_gather(xs_ref, [jnp.minimum(i + 1, N - 1)])),
            mg)
        ycand = jnp.where(
            r < N,
            _s2f(plsc.load_gather(ys_ref, [jnp.minimum(r, N - 1)])),
            mg)
        nxt = jnp.minimum(xnext, ycand)
        cx = (i + 1 - r).astype(jnp.float32) * inv_n
        acc = acc + cx * cx * (nxt - _s2f(xv))
        return (acc, rc)

    acc, _ = _body
    return acc


def _vsqrt(v):
    g = lax.shift_right_logical(plsc.bitcast(v, jnp.int32),
                                jnp.full((L,), 1, jnp.int32))
    y = plsc.bitcast(g + jnp.int32(0x1FBD1DF5), jnp.float32)
    for _ in range(4):
        y = jnp.float32(0.5) * (y + v / y)
    return jnp.where(v > 0, y, jnp.float32(0.0))


@functools.lru_cache(maxsize=None)
def _build():
    mesh = plsc.VectorSubcoreMesh(core_axis_name="c", subcore_axis_name="s")

    @functools.partial(
        pl.kernel,
        out_type=jax.ShapeDtypeStruct((B, L), jnp.float32),
        mesh=mesh,
        compiler_params=pltpu.CompilerParams(needs_layout_passes=False),
        scratch_types=[
            pltpu.VMEM((N,), jnp.int32),
            pltpu.VMEM((N,), jnp.int32),
            pltpu.VMEM((N,), jnp.int32),
            pltpu.VMEM((N,), jnp.int32),
            pltpu.VMEM((NBINS,), jnp.int32),
            pltpu.VMEM((N,), jnp.int32),
            pltpu.VMEM((N,), jnp.int32),
            pltpu.VMEM((RBINS,), jnp.int32),
            pltpu.VMEM((L,), jnp.float32),
        ],
    )
    def dist_kernel(x_hbm, y_hbm, out_hbm, xa, xb, ya, yb, hist, pack, dest, rbins, res):
        wid = lax.axis_index("s") * NC + lax.axis_index("c")
        zeros = jnp.zeros((L,), jnp.int32)

        def row_body(rr, _):
            row = wid * ROWS_PER_W + rr
            pltpu.sync_copy(x_hbm.at[row], xa)
            pltpu.sync_copy(y_hbm.at[row], ya)
            _radix_sort(ya, yb, hist, pack, dest)
            _radix_sort(xa, xb, hist, pack, dest)
            xs, ys = xb, yb

            @plsc.parallel_loop(0, RBINS // L, unroll=4)
            def _zr(i):
                rbins[pl.ds(i * L, L)] = zeros

            ms = jnp.maximum(jnp.max(xs[pl.ds(N - L, L)]),
                             jnp.max(ys[pl.ds(N - L, L)]))
            mg = _s2f(jnp.full((L,), ms, jnp.int32))
            acc = jnp.zeros((L,), jnp.float32)
            acc = _y_phase(xs, ys, rbins, hist, mg, acc)
            acc = _x_phase(xs, ys, rbins, mg, acc)
            res[...] = _vsqrt(jnp.full((L,), jnp.sum(acc), jnp.float32))
            pltpu.sync_copy(res, out_hbm.at[row])
            return 0

        lax.fori_loop(0, ROWS_PER_W, row_body, 0)

    return dist_kernel


def kernel(x_values, y_values):
    xi = lax.bitcast_convert_type(x_values, jnp.int32)
    yi = lax.bitcast_convert_type(y_values, jnp.int32)
    return _build()(xi, yi)[:, 0]


# speedup vs baseline: 2.8521x; 1.0059x over previous
import functools

import jax
import jax.numpy as jnp
from jax import lax
from jax.experimental import pallas as pl
from jax.experimental.pallas import tpu as pltpu
from jax.experimental.pallas import tpu_sc as plsc

B = 64
N = 16384
L = 16
NV = N // L
NBINS = 2048
RBINS = N + L
NC = 2
NS = 16
ROWS_PER_W = B // (NC * NS)


def _lane():
    return lax.iota(jnp.int32, L)


def _f2s(bits):
    return jnp.where(bits >= 0, bits, bits ^ jnp.int32(0x7FFFFFFF))


def _s2f(s):
    return plsc.bitcast(jnp.where(s >= 0, s, s ^ jnp.int32(0x7FFFFFFF)),
                        jnp.float32)


def _take(x, idx):
    return jnp.take_along_axis(x, idx, axis=0)


def _radix_sort(src_ref, tmp_ref, hist, pack, dest):
    ones = jnp.ones((L,), jnp.int32)
    zeros = jnp.zeros((L,), jnp.int32)

    bufs = [src_ref, tmp_ref]
    passes = [(0, 0x7FF, 0), (11, 0x7FF, 0), (22, 0x3FF, 0x200)]
    for p, (shift, dmask, flip) in enumerate(passes):
        a, b = bufs[p % 2], bufs[(p + 1) % 2]
        sh = jnp.full((L,), shift, jnp.int32)

        def keyfn(keys):
            return _f2s(keys) if p == 0 else keys

        def digit(keys):
            d = lax.shift_right_logical(keys, sh) & jnp.int32(dmask)
            return d ^ jnp.int32(flip) if flip else d

        @plsc.parallel_loop(0, NBINS // L, unroll=4)
        def _zero(i):
            hist[pl.ds(i * L, L)] = zeros

        @plsc.parallel_loop(0, NV, unroll=4)
        def _pre(v):
            sl = pl.ds(v * L, L)
            dig = digit(keyfn(a[sl]))
            cnt, lastm = plsc.scan_count(dig)
            plsc.addupdate_scatter(hist, [dig], cnt, mask=lastm)
            pack[sl] = (dig | ((cnt - 1) << 11)
                        | (jnp.where(lastm, 1, 0) << 15))

        @plsc.parallel_loop(0, NBINS // L, unroll=4,
                            carry=jnp.zeros((L,), jnp.int32))
        def _scan(i, carry):
            sl = pl.ds(i * L, L)
            h = hist[sl]
            c = plsc.cumsum(h)
            hist[sl] = carry + c - h
            return carry + jnp.full((L,), jnp.sum(h), jnp.int32)

        def dloop(i, _):
            for k in range(2):
                sl = pl.ds((2 * i + k) * L, L)
                pk = pack[sl]
                dig = pk & jnp.int32(0x7FF)
                occ = lax.shift_right_logical(
                    pk, jnp.full((L,), 11, jnp.int32)) & jnp.int32(0xF)
                lastm = pk >= jnp.int32(1 << 15)
                base = plsc.load_gather(hist, [dig])
                dest[sl] = base + occ
                plsc.addupdate_scatter(hist, [dig], occ + 1, mask=lastm)
            return 0

        lax.fori_loop(0, NV // 2, dloop, 0)

        @plsc.parallel_loop(0, NV, unroll=4)
        def _scatter(v):
            sl = pl.ds(v * L, L)
            plsc.store_scatter(b, [dest[sl]], keyfn(a[sl]))


def _y_phase(xs_ref, ys_ref, rbins_ref, xends_ref, mg, acc):
    lane = _lane()
    inv_n = jnp.float32(1.0 / N)

    @plsc.parallel_loop(0, NV, unroll=4, carry=acc)
    def _body(v, acc):
        j = v * L + lane
        yv = ys_ref[pl.ds(v * L, L)]
        d = (lax.shift_right_logical(yv, jnp.full((L,), 22, jnp.int32))
             & jnp.int32(0x3FF)) ^ jnp.int32(0x200)
        hi = plsc.load_gather(xends_ref, [d])
        lo = jnp.where(d > 0,
                       plsc.load_gather(xends_ref, [jnp.maximum(d - 1, 0)]),
                       0)
        for _ in range(15):
            mid = (lo + hi) >> 1
            val = plsc.load_gather(xs_ref, [jnp.minimum(mid, N - 1)])
            pred = val <= yv
            lo = jnp.where(pred, mid + 1, lo)
            hi = jnp.where(pred, hi, mid)
        q = lo
        ynext = jnp.where(
            j < N - 1,
            _s2f(plsc.load_gather(ys_ref, [jnp.minimum(j + 1, N - 1)])),
            mg)
        xcand = jnp.where(
            q < N,
            _s2f(plsc.load_gather(xs_ref, [jnp.minimum(q, N - 1)])),
            mg)
        nxt = jnp.minimum(ynext, xcand)
        cy = (q - (j + 1)).astype(jnp.float32) * inv_n
        acc = acc + cy * cy * (nxt - _s2f(yv))
        qprev = _take(q, jnp.maximum(lane - 1, 0))
        start = (lane == 0) | (q != qprev)
        startpos = plsc.cummax(jnp.where(start, lane, 0))
        cnt = lane - startpos + 1
        qnext = _take(q, jnp.minimum(lane + 1, L - 1))
        is_last = (lane == L - 1) | (q != qnext)
        plsc.addupdate_scatter(rbins_ref, [q], cnt, mask=is_last)
        return acc

    return _body


def _x_phase(xs_ref, ys_ref, rbins_ref, mg, acc):
    lane = _lane()
    inv_n = jnp.float32(1.0 / N)

    @plsc.parallel_loop(0, NV, unroll=4,
                        carry=(acc, jnp.zeros((L,), jnp.int32)))
    def _body(v, carry):
        acc, rc = carry
        i = v * L + lane
        cnts = rbins_ref[pl.ds(v * L, L)]
        r = plsc.cumsum(cnts) + rc
        rc = rc + jnp.full((L,), jnp.sum(cnts), jnp.int32)
        xv = xs_ref[pl.ds(v * L, L)]
        xnext = jnp.where(
            i < N - 1,
            _s2f(plsc.load_gather(xs_ref, [jnp.minimum(i + 1, N - 1)])),
            mg)
        ycand = jnp.where(
            r < N,
            _s2f(plsc.load_gather(ys_ref, [jnp.minimum(r, N - 1)])),
            mg)
        nxt = jnp.minimum(xnext, ycand)
        cx = (i + 1 - r).astype(jnp.float32) * inv_n
        acc = acc + cx * cx * (nxt - _s2f(xv))
        return (acc, rc)

    acc, _ = _body
    return acc


def _vsqrt(v):
    g = lax.shift_right_logical(plsc.bitcast(v, jnp.int32),
                                jnp.full((L,), 1, jnp.int32))
    y = plsc.bitcast(g + jnp.int32(0x1FBD1DF5), jnp.float32)
    for _ in range(4):
        y = jnp.float32(0.5) * (y + v / y)
    return jnp.where(v > 0, y, jnp.float32(0.0))


@functools.lru_cache(maxsize=None)
def _build():
    mesh = plsc.VectorSubcoreMesh(core_axis_name="c", subcore_axis_name="s")

    @functools.partial(
        pl.kernel,
        out_type=jax.ShapeDtypeStruct((B, L), jnp.float32),
        mesh=mesh,
        compiler_params=pltpu.CompilerParams(needs_layout_passes=False),
        scratch_types=[
            pltpu.VMEM((N,), jnp.int32),
            pltpu.VMEM((N,), jnp.int32),
            pltpu.VMEM((N,), jnp.int32),
            pltpu.VMEM((N,), jnp.int32),
            pltpu.VMEM((NBINS,), jnp.int32),
            pltpu.VMEM((N,), jnp.int32),
            pltpu.VMEM((N,), jnp.int32),
            pltpu.VMEM((RBINS,), jnp.int32),
            pltpu.VMEM((L,), jnp.float32),
        ],
    )
    def dist_kernel(x_hbm, y_hbm, out_hbm, xa, xb, ya, yb, hist, pack, dest, rbins, res):
        wid = lax.axis_index("s") * NC + lax.axis_index("c")
        zeros = jnp.zeros((L,), jnp.int32)

        def row_body(rr, _):
            row = wid * ROWS_PER_W + rr
            pltpu.sync_copy(x_hbm.at[row], xa)
            pltpu.sync_copy(y_hbm.at[row], ya)
            _radix_sort(ya, yb, hist, pack, dest)
            _radix_sort(xa, xb, hist, pack, dest)
            xs, ys = xb, yb

            @plsc.parallel_loop(0, RBINS // L, unroll=4)
            def _zr(i):
                rbins[pl.ds(i * L, L)] = zeros

            ms = jnp.maximum(jnp.max(xs[pl.ds(N - L, L)]),
                             jnp.max(ys[pl.ds(N - L, L)]))
            mg = _s2f(jnp.full((L,), ms, jnp.int32))
            acc = jnp.zeros((L,), jnp.float32)
            acc = _y_phase(xs, ys, rbins, hist, mg, acc)
            acc = _x_phase(xs, ys, rbins, mg, acc)
            res[...] = _vsqrt(jnp.full((L,), jnp.sum(acc), jnp.float32))
            pltpu.sync_copy(res, out_hbm.at[row])
            return 0

        lax.fori_loop(0, ROWS_PER_W, row_body, 0)

    return dist_kernel


def kernel(x_values, y_values):
    xi = lax.bitcast_convert_type(x_values, jnp.int32)
    yi = lax.bitcast_convert_type(y_values, jnp.int32)
    return _build()(xi, yi)[:, 0]
